# baseline (speedup 1.0000x reference)
import functools

import jax
import jax.numpy as jnp
from jax import lax
from jax.experimental import pallas as pl
from jax.experimental.pallas import tpu as pltpu
from jax.experimental.pallas import tpu_sc as plsc

_THRESH = 0.5
_B = 64
_N = 32768
_L = 16
_CHUNKS = _N // _L
_G = 16
_SEG = 16384
_NSEG = _N // _SEG
_SGRP = _SEG // _L // _G
_FILL_TOP = _CHUNKS - 11 * _SGRP
_R = 8


def _tec_body(probs_hbm, out_hbm, ring0, ring1, ov0, ov1,
              sin0, sin1, sout0, sout1):
    cid = lax.axis_index("c")
    sid = lax.axis_index("s")
    wid = sid * 2 + cid
    neg1 = jnp.full((_L,), -1, jnp.int32)
    zeros = jnp.zeros((_L,), jnp.int32)
    lane = lax.iota(jnp.int32, _L)
    nsplat = jnp.full((_L,), _N, jnp.int32)

    rows = [wid * 2, wid * 2 + 1]
    rings = [ring0, ring1]
    sins = [sin0, sin1]
    ovs = [ov0, ov1]
    souts = [sout0, sout1]

    def in_copy(q):
        r = rows[q // _NSEG]
        s = q % _NSEG
        return pltpu.make_async_copy(
            probs_hbm.at[r, pl.ds(s * _SEG, _SEG)],
            rings[q % 2], sins[q % 2])

    def compact_segment(buf, ov, carry, fill):
        def cbody(it, carry):
            blane, idxs, pend = carry
            i0 = it * _G
            if fill:
                for k in range(11):
                    fc = (_CHUNKS - 1) - 11 * it - k
                    ov[pl.ds(fc * _L, _L)] = neg1
            for pp, pv in zip(pend[:_R], pend[_R:]):
                plsc.store_scatter(ov, [pp], pv)
            vs = [buf[pl.ds((i0 + g) * _L, _L)] for g in range(_G)]
            ms = [v >= _THRESH for v in vs]
            cands = [
                plsc.bitcast(
                    jnp.where(m,
                              idxs + jnp.full((_L,), g * _L, jnp.int32),
                              neg1),
                    jnp.uint32)
                for g, m in enumerate(ms)
            ]
            sorted_vals = [plsc.bitcast(lax.sort(c, dimension=0), jnp.int32)
                           for c in cands]
            cnts = [plsc.all_reduce_population_count(m) for m in ms]
            blanes = [blane]
            for g in range(_G):
                blanes.append(blanes[g] + cnts[g])
            for g in range(_G - _R):
                plsc.store_scatter(ov, [blanes[g]], sorted_vals[g])
            newpend = (tuple(blanes[g] for g in range(_G - _R, _G))
                       + tuple(sorted_vals[g] for g in range(_G - _R, _G)))
            return (blanes[_G], idxs + jnp.full((_L,), _G * _L, jnp.int32),
                    newpend)

        return lax.fori_loop(0, _SGRP, cbody, carry)

    def tail_fill(ov, count_splat):
        start = count_splat & jnp.full((_L,), ~(_L - 1), jnp.int32)
        tpos = start + lane
        tmask = jnp.logical_and(tpos >= count_splat, tpos < nsplat)
        plsc.store_scatter(ov, [tpos], neg1, mask=tmask)
        count = count_splat[0]
        kc1 = count // _L + 1
        sg = (kc1 + 15) // 16
        for k in range(15):
            ppos = (kc1 + k) * _L + lane
            pmask = jnp.logical_and(ppos < sg * 16 * _L, ppos < nsplat)
            plsc.store_scatter(ov, [ppos], neg1, mask=pmask)

        def fbody(j, carry):
            for k in range(16):
                ov[pl.ds((j * 16 + k) * _L, _L)] = neg1
            return carry

        lax.fori_loop(sg, _FILL_TOP // 16, fbody, 0)

    handles = {}
    for q in range(2):
        handles[q] = in_copy(q)
        handles[q].start()
    out_handles = []
    top = jnp.full((_L,), _N - _L, jnp.int32) + lane
    for ri in range(2):
        carry = (lane, lane, (top,) * _R + (neg1,) * _R)
        for s in range(_NSEG):
            q = ri * _NSEG + s
            handles[q].wait()
            carry = compact_segment(rings[q % 2], ovs[ri], carry, s == 0)
            if q + 2 < 2 * _NSEG:
                handles[q + 2] = in_copy(q + 2)
                handles[q + 2].start()
        for pp, pv in zip(carry[2][:_R], carry[2][_R:]):
            plsc.store_scatter(ovs[ri], [pp], pv)
        tail_fill(ovs[ri], carry[0] - lane)
        h = pltpu.make_async_copy(ovs[ri], out_hbm.at[rows[ri]], souts[ri])
        h.start()
        out_handles.append(h)
    for h in out_handles:
        h.wait()


_fn_cache = []


def _get_fn():
    if not _fn_cache:
        mesh = plsc.VectorSubcoreMesh(core_axis_name="c",
                                      subcore_axis_name="s")
        fn = functools.partial(
            pl.kernel,
            out_type=jax.ShapeDtypeStruct((_B, _N), jnp.int32),
            mesh=mesh,
            scratch_types=[
                pltpu.VMEM((_SEG,), jnp.float32),
                pltpu.VMEM((_SEG,), jnp.float32),
                pltpu.VMEM((_N,), jnp.int32),
                pltpu.VMEM((_N,), jnp.int32),
                pltpu.SemaphoreType.DMA,
                pltpu.SemaphoreType.DMA,
                pltpu.SemaphoreType.DMA,
                pltpu.SemaphoreType.DMA,
            ],
            compiler_params=pltpu.CompilerParams(needs_layout_passes=False),
        )(_tec_body)
        _fn_cache.append(fn)
    return _fn_cache[0]


def kernel(probs):
    return _get_fn()(probs)


# speedup vs baseline: 1.0197x; 1.0197x over previous
import functools

import jax
import jax.numpy as jnp
from jax import lax
from jax.experimental import pallas as pl
from jax.experimental.pallas import tpu as pltpu
from jax.experimental.pallas import tpu_sc as plsc

_THRESH = 0.5
_B = 64
_N = 32768
_L = 16
_CHUNKS = _N // _L
_G = 16
_SEG = 16384
_NSEG = _N // _SEG
_SGRP = _SEG // _L // _G
_FILL_TOP = _CHUNKS - 11 * _SGRP


def _tec_body(probs_hbm, out_hbm, ring0, ring1, ov0, ov1,
              sin0, sin1, sout0, sout1):
    cid = lax.axis_index("c")
    sid = lax.axis_index("s")
    wid = sid * 2 + cid
    neg1 = jnp.full((_L,), -1, jnp.int32)
    zeros = jnp.zeros((_L,), jnp.int32)
    lane = lax.iota(jnp.int32, _L)
    nsplat = jnp.full((_L,), _N, jnp.int32)

    rows = [wid * 2, wid * 2 + 1]
    rings = [ring0, ring1]
    sins = [sin0, sin1]
    ovs = [ov0, ov1]
    souts = [sout0, sout1]

    def in_copy(q):
        r = rows[q // _NSEG]
        s = q % _NSEG
        return pltpu.make_async_copy(
            probs_hbm.at[r, pl.ds(s * _SEG, _SEG)],
            rings[q % 2], sins[q % 2])

    def compact_segment(buf, ov, carry, fill):
        def cbody(it, carry):
            blane, idxs = carry
            i0 = it * _G
            if fill:
                for k in range(11):
                    fc = (_CHUNKS - 1) - 11 * it - k
                    ov[pl.ds(fc * _L, _L)] = neg1
            vs = [buf[pl.ds((i0 + g) * _L, _L)] for g in range(_G)]
            ms = [v >= _THRESH for v in vs]
            cands = [
                plsc.bitcast(
                    jnp.where(m,
                              idxs + jnp.full((_L,), g * _L, jnp.int32),
                              neg1),
                    jnp.uint32)
                for g, m in enumerate(ms)
            ]
            sorted_vals = [plsc.bitcast(lax.sort(c, dimension=0), jnp.int32)
                           for c in cands]
            cnts = [plsc.all_reduce_population_count(m) for m in ms]
            blanes = [blane]
            for g in range(_G):
                blanes.append(blanes[g] + cnts[g])
            for g in range(_G):
                plsc.store_scatter(ov, [blanes[g]], sorted_vals[g])
            return (blanes[_G], idxs + jnp.full((_L,), _G * _L, jnp.int32))

        return lax.fori_loop(0, _SGRP, cbody, carry)

    def tail_fill(ov, count_splat):
        start = count_splat & jnp.full((_L,), ~(_L - 1), jnp.int32)
        tpos = start + lane
        tmask = jnp.logical_and(tpos >= count_splat, tpos < nsplat)
        plsc.store_scatter(ov, [tpos], neg1, mask=tmask)
        count = count_splat[0]
        kc1 = count // _L + 1
        sg = (kc1 + 15) // 16
        for k in range(15):
            ppos = (kc1 + k) * _L + lane
            pmask = jnp.logical_and(ppos < sg * 16 * _L, ppos < nsplat)
            plsc.store_scatter(ov, [ppos], neg1, mask=pmask)

        def fbody(j, carry):
            for k in range(16):
                ov[pl.ds((j * 16 + k) * _L, _L)] = neg1
            return carry

        lax.fori_loop(sg, _FILL_TOP // 16, fbody, 0)

    handles = {}
    for q in range(2):
        handles[q] = in_copy(q)
        handles[q].start()
    out_handles = []
    for ri in range(2):
        carry = (lane, lane)
        for s in range(_NSEG):
            q = ri * _NSEG + s
            handles[q].wait()
            carry = compact_segment(rings[q % 2], ovs[ri], carry, s == 0)
            if q + 2 < 2 * _NSEG:
                handles[q + 2] = in_copy(q + 2)
                handles[q + 2].start()
        tail_fill(ovs[ri], carry[0] - lane)
        h = pltpu.make_async_copy(ovs[ri], out_hbm.at[rows[ri]], souts[ri])
        h.start()
        out_handles.append(h)
    for h in out_handles:
        h.wait()


_fn_cache = []


def _get_fn():
    if not _fn_cache:
        mesh = plsc.VectorSubcoreMesh(core_axis_name="c",
                                      subcore_axis_name="s")
        fn = functools.partial(
            pl.kernel,
            out_type=jax.ShapeDtypeStruct((_B, _N), jnp.int32),
            mesh=mesh,
            scratch_types=[
                pltpu.VMEM((_SEG,), jnp.float32),
                pltpu.VMEM((_SEG,), jnp.float32),
                pltpu.VMEM((_N,), jnp.int32),
                pltpu.VMEM((_N,), jnp.int32),
                pltpu.SemaphoreType.DMA,
                pltpu.SemaphoreType.DMA,
                pltpu.SemaphoreType.DMA,
                pltpu.SemaphoreType.DMA,
            ],
            compiler_params=pltpu.CompilerParams(needs_layout_passes=False),
        )(_tec_body)
        _fn_cache.append(fn)
    return _fn_cache[0]


def kernel(probs):
    return _get_fn()(probs)
